# Initial kernel scaffold; baseline (speedup 1.0000x reference)
import jax
import jax.numpy as jnp
from jax.experimental import pallas as pl


def kernel(node_sca, node_vec, edge_feats, edge_index, W1, b1, W2, b2):
    raise NotImplementedError("write your pallas kernel here")


# speedup vs baseline: 11.4670x; 11.4670x over previous
import functools
import math

import jax
import jax.numpy as jnp
from jax.experimental import pallas as pl

HIDDEN = 128
NF = 128
ROW = 4 * NF


def _dense_body(agg_ref, sca_ref, vec_ref, w1t_ref, b1_ref, w2t_ref, b2_ref,
                sca_out_ref, vec_out_ref):
    agg = agg_ref[...]
    s = (agg[:, 0 * NF:1 * NF] + agg[:, 1 * NF:2 * NF]
         + agg[:, 2 * NF:3 * NF] + agg[:, 3 * NF:4 * NF])
    h = jnp.dot(s, w1t_ref[...], preferred_element_type=jnp.float32) + b1_ref[...]
    h = jnp.maximum(h, 0.0) + jnp.log1p(jnp.exp(-jnp.abs(h))) - math.log(2.0)
    h = jnp.dot(h, w2t_ref[...], preferred_element_type=jnp.float32) + b2_ref[...]
    sca_out_ref[...] = sca_ref[...] + h
    vec_out_ref[...] = vec_ref[...] + agg[:, NF:]


def _dense_phase(agg, node_sca, node_vec_flat, W1, b1, W2, b2):
    n = node_sca.shape[0]
    blk = 2000
    grid = (n // blk,)
    fixed = lambda i: (0, 0)
    out = pl.pallas_call(
        _dense_body,
        grid=grid,
        in_specs=[
            pl.BlockSpec((blk, ROW), lambda i: (i, 0)),
            pl.BlockSpec((blk, HIDDEN), lambda i: (i, 0)),
            pl.BlockSpec((blk, 3 * NF), lambda i: (i, 0)),
            pl.BlockSpec((HIDDEN, NF), fixed),
            pl.BlockSpec((1, HIDDEN), fixed),
            pl.BlockSpec((HIDDEN, HIDDEN), fixed),
            pl.BlockSpec((1, HIDDEN), fixed),
        ],
        out_specs=[
            pl.BlockSpec((blk, HIDDEN), lambda i: (i, 0)),
            pl.BlockSpec((blk, 3 * NF), lambda i: (i, 0)),
        ],
        out_shape=[
            jax.ShapeDtypeStruct((n, HIDDEN), jnp.float32),
            jax.ShapeDtypeStruct((n, 3 * NF), jnp.float32),
        ],
    )(agg, node_sca, node_vec_flat, W1.T, b1[None, :], W2.T, b2[None, :])
    return out


def kernel(node_sca, node_vec, edge_feats, edge_index, W1, b1, W2, b2):
    n = node_sca.shape[0]
    e = edge_feats.shape[0]
    dst = edge_index[1].astype(jnp.int32)
    agg = jax.ops.segment_sum(edge_feats.reshape(e, ROW), dst, num_segments=n)
    sca_out, vec_out = _dense_phase(
        agg, node_sca, node_vec.reshape(n, 3 * NF), W1, b1, W2, b2)
    return (sca_out, vec_out.reshape(n, 3, NF))
